# Initial kernel scaffold; baseline (speedup 1.0000x reference)
import jax
import jax.numpy as jnp
from jax.experimental import pallas as pl


def kernel(x, edge_index, W0, b0, g0, be0, W1, b1, g1, be1, W2, b2):
    raise NotImplementedError("write your pallas kernel here")


# speedup vs baseline: 4.5880x; 4.5880x over previous
import functools

import jax
import jax.numpy as jnp
from jax import lax
from jax.experimental import pallas as pl
from jax.experimental.pallas import tpu as pltpu
from jax.experimental.pallas import tpu_sc as plsc

N = 10000
NP = 10240
D = 128
E = 320000
EPS = 1e-5

NC = 2
NS = 16
NH = NP // NC
NACC = NH + 128
ERS = 2560
EP = ERS * 128
RPW = ERS // NS
GRP = 8
NGRP = RPW // GRP
WAVE = 4
RPT = NACC // NS
DRT = NP // NS

_mesh = plsc.VectorSubcoreMesh(
    core_axis_name="c", subcore_axis_name="s", num_cores=NC, num_subcores=NS)


@functools.partial(
    pl.kernel,
    out_type=jax.ShapeDtypeStruct((NC, NP), jnp.float32),
    mesh=_mesh,
    scratch_types=[
        pltpu.VMEM((DRT,), jnp.float32),
        pltpu.VMEM((128,), jnp.float32),
        pltpu.VMEM((GRP, 128), jnp.int32),
        pltpu.VMEM_SHARED((NP,), jnp.float32),
    ],
)
def _deg_kernel(dst_hbm, out_hbm, zbuf, ones_v, didx, acc):
    c = lax.axis_index("c")
    s = lax.axis_index("s")
    w = c * NS + s

    def _zero(i, carry):
        zbuf[pl.ds(i * 16, 16)] = jnp.zeros((16,), jnp.float32)
        return carry
    lax.fori_loop(0, DRT // 16, _zero, 0)
    for i in range(8):
        ones_v[pl.ds(i * 16, 16)] = jnp.ones((16,), jnp.float32)
    pltpu.sync_copy(zbuf, acc.at[pl.ds(s * DRT, DRT)])
    plsc.subcore_barrier()

    r0 = w * (RPW // 2)

    def _grp(g, carry):
        pltpu.sync_copy(dst_hbm.at[pl.ds(r0 + g * GRP, GRP)], didx)
        for j in range(GRP):
            pltpu.sync_copy(ones_v, acc.at[didx.at[j]], add=True)
        return carry
    lax.fori_loop(0, RPW // 2 // GRP, _grp, 0)

    plsc.subcore_barrier()
    pltpu.sync_copy(acc.at[pl.ds(s * DRT, DRT)],
                    out_hbm.at[c, pl.ds(s * DRT, DRT)])


@functools.partial(
    pl.kernel,
    out_type=jax.ShapeDtypeStruct((NC, NH, D), jnp.float32),
    mesh=_mesh,
    scratch_types=[
        pltpu.VMEM((WAVE * 128, D), jnp.float32),
        pltpu.VMEM((GRP, 128), jnp.int32),
        pltpu.VMEM((GRP, 128), jnp.int32),
        pltpu.VMEM((GRP, 128), jnp.int32),
        pltpu.VMEM_SHARED((NACC, D), jnp.float32),
        pltpu.SemaphoreType.DMA,
    ],
)
def _spmm_kernel(src_hbm, dst_hbm, table_hbm, out_hbm,
                 rows, sidx, didx, dloc, acc, sem):
    c = lax.axis_index("c")
    s = lax.axis_index("s")
    lo = c * NH

    def _zero(k, carry):
        rows[k // 8, pl.ds((k % 8) * 16, 16)] = jnp.zeros((16,), jnp.float32)
        return carry
    lax.fori_loop(0, RPT * 8, _zero, 0)
    pltpu.sync_copy(rows.at[pl.ds(0, RPT)], acc.at[pl.ds(s * RPT, RPT)])
    plsc.subcore_barrier()

    r0 = s * RPW

    def _grp(g, carry):
        base = r0 + g * GRP
        pltpu.sync_copy(src_hbm.at[pl.ds(base, GRP)], sidx)
        pltpu.sync_copy(dst_hbm.at[pl.ds(base, GRP)], didx)
        for j in range(GRP):
            for k in range(8):
                dv = didx[j, pl.ds(k * 16, 16)] - lo
                keep = (dv >= 0) & (dv < NH)
                dloc[j, pl.ds(k * 16, 16)] = jnp.where(keep, dv, NH)
        for h in range(GRP // WAVE):
            cps = [pltpu.async_copy(table_hbm.at[sidx.at[h * WAVE + j]],
                                    rows.at[pl.ds(j * 128, 128)], sem)
                   for j in range(WAVE)]
            for cp in cps:
                cp.wait()
            for j in range(WAVE):
                pltpu.sync_copy(rows.at[pl.ds(j * 128, 128)],
                                acc.at[dloc.at[h * WAVE + j]], add=True)
        return carry
    lax.fori_loop(0, NGRP, _grp, 0)

    plsc.subcore_barrier()
    pltpu.sync_copy(acc.at[pl.ds(s * (NH // NS), NH // NS)],
                    out_hbm.at[c, pl.ds(s * (NH // NS), NH // NS)])


_BLK = 1024
_GRID = NP // _BLK

def _dot(a, b):
    return lax.dot_general(a, b, (((1,), (0,)), ((), ())),
                           precision=lax.Precision.HIGHEST,
                           preferred_element_type=jnp.float32)


def _k0_body(x_ref, d_ref, w_ref, o_ref):
    o_ref[...] = _dot(d_ref[...] * x_ref[...], w_ref[...])


def _kmid_body(acc_ref, g_ref, d_ref, w_ref, t_ref, c1_ref, o_ref):
    u = d_ref[...] * (acc_ref[...] + g_ref[...])
    h = jnp.maximum(t_ref[...] * u + c1_ref[...], 0.0)
    o_ref[...] = _dot(d_ref[...] * h, w_ref[...])


def _k3_body(acc_ref, g_ref, d_ref, b_ref, o_ref):
    o_ref[...] = d_ref[...] * (acc_ref[...] + g_ref[...]) + b_ref[...]


_row_spec = pl.BlockSpec((_BLK, D), lambda i: (i, 0))
_mat_spec = pl.BlockSpec((D, D), lambda i: (0, 0))
_vec_spec = pl.BlockSpec((1, D), lambda i: (0, 0))
_out_sds = jax.ShapeDtypeStruct((NP, D), jnp.float32)

_k0 = pl.pallas_call(
    _k0_body, grid=(_GRID,),
    in_specs=[_row_spec, _row_spec, _mat_spec],
    out_specs=_row_spec, out_shape=_out_sds)

_kmid = pl.pallas_call(
    _kmid_body, grid=(_GRID,),
    in_specs=[_row_spec, _row_spec, _row_spec, _mat_spec, _vec_spec,
              _vec_spec],
    out_specs=_row_spec, out_shape=_out_sds)

_k3 = pl.pallas_call(
    _k3_body, grid=(_GRID,),
    in_specs=[_row_spec, _row_spec, _row_spec, _vec_spec],
    out_specs=_row_spec, out_shape=_out_sds)


def kernel(x, edge_index, W0, b0, g0, be0, W1, b1, g1, be1, W2, b2):
    src = edge_index[0]
    dst = edge_index[1]
    pad = EP - E
    srcp = jnp.concatenate(
        [src, jnp.zeros((pad,), src.dtype)]).reshape(ERS, 128)
    dstp = jnp.concatenate(
        [dst, jnp.full((pad,), N, dst.dtype)]).reshape(ERS, 128)
    xp = jnp.concatenate([x, jnp.zeros((NP - N, D), x.dtype)], axis=0)

    degs = _deg_kernel(dstp)
    dinv = lax.rsqrt(degs[0] + degs[1] + 1.0)
    dinv2d = jnp.broadcast_to(dinv[:, None], (NP, D))

    cbn = (1.0 + EPS) ** -0.5
    t0 = (g0 * cbn).reshape(1, D)
    c10 = (t0[0] * b0 + be0).reshape(1, D)
    t1 = (g1 * cbn).reshape(1, D)
    c11 = (t1[0] * b1 + be1).reshape(1, D)
    b2r = b2.reshape(1, D)

    G0 = _k0(xp, dinv2d, W0)
    acc = _spmm_kernel(srcp, dstp, G0).reshape(NP, D)
    G1 = _kmid(acc, G0, dinv2d, W1, t0, c10)
    acc = _spmm_kernel(srcp, dstp, G1).reshape(NP, D)
    G2 = _kmid(acc, G1, dinv2d, W2, t1, c11)
    acc = _spmm_kernel(srcp, dstp, G2).reshape(NP, D)
    z = _k3(acc, G2, dinv2d, b2r)
    return z[:N]
